# baseline (speedup 1.0000x reference)
import functools

import jax
import jax.numpy as jnp
from jax import lax
from jax.experimental import pallas as pl
from jax.experimental.pallas import tpu as pltpu
from jax.experimental.pallas import tpu_sc as plsc

N_NODES = 10000
N_EDGES = 320000
D_FEAT = 128
HIDDEN = 16

NC, NS = 2, 16
NW = NC * NS
EPW = 10240
NEP = NW * EPW
EPC = 512
NCHK = EPW // EPC
NBUF = 4
NP = 10240
RPS = NP // NS

_mesh = plsc.VectorSubcoreMesh(core_axis_name="c", subcore_axis_name="s")
_sc_params = pltpu.CompilerParams(use_tc_tiling_on_sc=False)


def _worker_id():
    c = lax.axis_index("c")
    s = lax.axis_index("s")
    return c, s, s * NC + c


@functools.partial(
    pl.kernel,
    out_type=jax.ShapeDtypeStruct((NC, NP), jnp.float32),
    mesh=_mesh,
    scratch_types=[
        pltpu.VMEM((EPW,), jnp.int32),
        pltpu.VMEM((EPC,), jnp.float32),
        pltpu.VMEM_SHARED((NP,), jnp.float32),
        pltpu.SemaphoreType.DMA,
    ],
    compiler_params=_sc_params,
)
def _deg_kernel(ei2_hbm, zd_hbm, out_hbm, didx_v, ones_v, acc_sh, sem):
    c, s, w = _worker_id()

    def fill_ones(i, _):
        ones_v[pl.ds(i * 16, 16)] = jnp.full((16,), 1.0, jnp.float32)
        return 0

    lax.fori_loop(0, EPC // 16, fill_ones, 0)
    pltpu.sync_copy(zd_hbm.at[pl.ds(s * RPS, RPS)],
                    acc_sh.at[pl.ds(s * RPS, RPS)])
    plsc.subcore_barrier()

    pltpu.async_copy(ei2_hbm.at[1, pl.ds(w * EPW, EPW)], didx_v, sem).wait()

    def step(k, _):
        pltpu.sync_copy(ones_v, acc_sh.at[didx_v.at[pl.ds(k * EPC, EPC)]],
                        add=True)
        return 0

    lax.fori_loop(0, NCHK, step, 0)
    plsc.subcore_barrier()
    pltpu.sync_copy(acc_sh.at[pl.ds(s * RPS, RPS)],
                    out_hbm.at[c, pl.ds(s * RPS, RPS)])


@functools.partial(
    pl.kernel,
    out_type=jax.ShapeDtypeStruct((NC, NP, D_FEAT), jnp.float32),
    mesh=_mesh,
    scratch_types=[
        pltpu.VMEM((EPW,), jnp.int32),
        pltpu.VMEM((EPW,), jnp.int32),
        pltpu.VMEM((EPC, HIDDEN), jnp.float32),
        pltpu.VMEM((EPC, HIDDEN), jnp.float32),
        pltpu.VMEM((EPC, HIDDEN), jnp.float32),
        pltpu.VMEM((EPC, HIDDEN), jnp.float32),
        pltpu.VMEM_SHARED((NP, HIDDEN), jnp.float32),
        pltpu.SemaphoreType.DMA,
        pltpu.SemaphoreType.DMA,
        pltpu.SemaphoreType.DMA,
        pltpu.SemaphoreType.DMA,
        pltpu.SemaphoreType.DMA,
    ],
    compiler_params=_sc_params,
)
def _agg_kernel(g_hbm, ei2_hbm, z_hbm, out_hbm,
                sidx_v, didx_v, b0, b1, b2, b3, acc_sh,
                sem_i, s0, s1, s2, s3):
    c, s, w = _worker_id()
    bufs = (b0, b1, b2, b3)
    sems = (s0, s1, s2, s3)

    pltpu.sync_copy(z_hbm.at[pl.ds(s * RPS, RPS)],
                    acc_sh.at[pl.ds(s * RPS, RPS)])
    plsc.subcore_barrier()

    cp_s = pltpu.async_copy(ei2_hbm.at[0, pl.ds(w * EPW, EPW)], sidx_v, sem_i)
    cp_d = pltpu.async_copy(ei2_hbm.at[1, pl.ds(w * EPW, EPW)], didx_v, sem_i)
    cp_s.wait()
    cp_d.wait()

    for b in range(NBUF):
        pltpu.async_copy(g_hbm.at[sidx_v.at[pl.ds(b * EPC, EPC)]],
                         bufs[b], sems[b])

    def it_body(it, _):
        for b in range(NBUF):
            k = it * NBUF + b
            pltpu.make_async_copy(g_hbm.at[sidx_v.at[pl.ds(k * EPC, EPC)]],
                                  bufs[b], sems[b]).wait()
            pltpu.sync_copy(bufs[b],
                            acc_sh.at[didx_v.at[pl.ds(k * EPC, EPC)]],
                            add=True)

            @pl.when(it < NCHK // NBUF - 1)
            def _():
                pltpu.async_copy(
                    g_hbm.at[sidx_v.at[pl.ds((k + NBUF) * EPC, EPC)]],
                    bufs[b], sems[b])

        return 0

    lax.fori_loop(0, NCHK // NBUF, it_body, 0)

    plsc.subcore_barrier()
    pltpu.sync_copy(acc_sh.at[pl.ds(s * RPS, RPS)],
                    out_hbm.at[c, pl.ds(s * RPS, RPS), pl.ds(0, HIDDEN)])


def _dinv(dp_ref):
    return lax.rsqrt(dp_ref[0, :N_NODES] + dp_ref[1, :N_NODES] + 1.0)


def _tc1_body(x_ref, w1_ref, dp_ref, g1_ref):
    h = jnp.dot(x_ref[...], w1_ref[...], preferred_element_type=jnp.float32)
    g1_ref[:, pl.ds(0, HIDDEN)] = h * _dinv(dp_ref)[:, None]


def _tc2_body(agg_ref, g1_ref, dp_ref, b1_ref, w2_ref, g2_ref):
    dinv = _dinv(dp_ref)
    a = (agg_ref[0, :N_NODES, :HIDDEN] + agg_ref[1, :N_NODES, :HIDDEN]
         + g1_ref[:, :HIDDEN])
    h = jnp.maximum(a * dinv[:, None] + b1_ref[...], 0.0)
    h2 = jnp.dot(h, w2_ref[...], preferred_element_type=jnp.float32)
    g2_ref[:, pl.ds(0, HIDDEN)] = h2 * dinv[:, None]


def _tc3_body(agg_ref, g2_ref, dp_ref, b2_ref, out_ref):
    dinv = _dinv(dp_ref)
    a = (agg_ref[0, :N_NODES, :HIDDEN] + agg_ref[1, :N_NODES, :HIDDEN]
         + g2_ref[:, :HIDDEN])
    o = a * dinv[:, None] + b2_ref[...]
    m = jnp.max(o, axis=1, keepdims=True)
    e = jnp.exp(o - m)
    out_ref[...] = (o - m) - jnp.log(jnp.sum(e, axis=1, keepdims=True))


_WIDE_F32 = jax.ShapeDtypeStruct((N_NODES, D_FEAT), jnp.float32)


def kernel(x, edge_index, W1, b1, W2, b2):
    ei = edge_index.astype(jnp.int32)
    npad = NEP - N_EDGES
    src8 = jnp.concatenate([ei[0] * 8, jnp.zeros((npad,), jnp.int32)])
    dstp = jnp.concatenate([ei[1], jnp.full((npad,), N_NODES, jnp.int32)])
    ei2 = jnp.stack([src8, dstp])
    zd = jnp.zeros((NP,), jnp.float32)
    z = jnp.zeros((NP, HIDDEN), jnp.float32)
    b1r = b1.reshape(1, HIDDEN)
    b2r = b2.reshape(1, HIDDEN)

    dp = _deg_kernel(ei2, zd)
    g1 = pl.pallas_call(_tc1_body, out_shape=_WIDE_F32)(x, W1, dp)
    agg1 = _agg_kernel(g1.reshape(N_NODES * 8, HIDDEN), ei2, z)
    g2 = pl.pallas_call(_tc2_body, out_shape=_WIDE_F32)(agg1, g1, dp, b1r, W2)
    agg2 = _agg_kernel(g2.reshape(N_NODES * 8, HIDDEN), ei2, z)
    out = pl.pallas_call(
        _tc3_body,
        out_shape=jax.ShapeDtypeStruct((N_NODES, HIDDEN), jnp.float32),
    )(agg2, g2, dp, b2r)
    return out


# speedup vs baseline: 1.6884x; 1.6884x over previous
import functools

import jax
import jax.numpy as jnp
from jax import lax
from jax.experimental import pallas as pl
from jax.experimental.pallas import tpu as pltpu
from jax.experimental.pallas import tpu_sc as plsc

N_NODES = 10000
N_EDGES = 320000
D_FEAT = 128
HIDDEN = 16

NC, NS = 2, 16
NW = NC * NS
EPW = 10240
NEP = NW * EPW
EPC = 512
NCHK = EPW // EPC
NBUF = 4
NP = 10240
RPS = NP // NS

_mesh = plsc.VectorSubcoreMesh(core_axis_name="c", subcore_axis_name="s")
_sc_params = pltpu.CompilerParams(use_tc_tiling_on_sc=False)


def _worker_id():
    c = lax.axis_index("c")
    s = lax.axis_index("s")
    return c, s, s * NC + c


@functools.partial(
    pl.kernel,
    out_type=jax.ShapeDtypeStruct((NC, NP), jnp.float32),
    mesh=_mesh,
    scratch_types=[
        pltpu.VMEM((EPW,), jnp.int32),
        pltpu.VMEM((EPC,), jnp.float32),
        pltpu.VMEM_SHARED((NP,), jnp.float32),
        pltpu.SemaphoreType.DMA,
    ],
    compiler_params=_sc_params,
)
def _deg_kernel(ei2_hbm, zd_hbm, out_hbm, didx_v, ones_v, acc_sh, sem):
    c, s, w = _worker_id()

    def fill_ones(i, _):
        ones_v[pl.ds(i * 16, 16)] = jnp.full((16,), 1.0, jnp.float32)
        return 0

    lax.fori_loop(0, EPC // 16, fill_ones, 0)
    pltpu.sync_copy(zd_hbm.at[pl.ds(s * RPS, RPS)],
                    acc_sh.at[pl.ds(s * RPS, RPS)])
    plsc.subcore_barrier()

    pltpu.async_copy(ei2_hbm.at[1, pl.ds(w * EPW, EPW)], didx_v, sem).wait()

    def step(k, _):
        pltpu.sync_copy(ones_v, acc_sh.at[didx_v.at[pl.ds(k * EPC, EPC)]],
                        add=True)
        return 0

    lax.fori_loop(0, NCHK, step, 0)
    plsc.subcore_barrier()
    pltpu.sync_copy(acc_sh.at[pl.ds(s * RPS, RPS)],
                    out_hbm.at[c, pl.ds(s * RPS, RPS)])


@functools.partial(
    pl.kernel,
    out_type=jax.ShapeDtypeStruct((NC, NP, D_FEAT), jnp.float32),
    mesh=_mesh,
    scratch_types=[
        pltpu.VMEM((EPW,), jnp.int32),
        pltpu.VMEM((EPW,), jnp.int32),
        pltpu.VMEM((EPC, HIDDEN), jnp.float32),
        pltpu.VMEM((EPC, HIDDEN), jnp.float32),
        pltpu.VMEM((EPC, HIDDEN), jnp.float32),
        pltpu.VMEM((EPC, HIDDEN), jnp.float32),
        pltpu.VMEM_SHARED((NP, HIDDEN), jnp.float32),
        pltpu.SemaphoreType.DMA,
        pltpu.SemaphoreType.DMA,
        pltpu.SemaphoreType.DMA,
        pltpu.SemaphoreType.DMA,
        pltpu.SemaphoreType.DMA,
    ],
    compiler_params=_sc_params,
)
def _agg_kernel(g_hbm, ei2_hbm, z_hbm, out_hbm,
                sidx_v, didx_v, b0, b1, b2, b3, acc_sh,
                sem_i, s0, s1, s2, s3):
    c, s, w = _worker_id()
    bufs = (b0, b1, b2, b3)
    sems = (s0, s1, s2, s3)

    pltpu.sync_copy(z_hbm.at[pl.ds(s * RPS, RPS)],
                    acc_sh.at[pl.ds(s * RPS, RPS)])
    plsc.subcore_barrier()

    cp_s = pltpu.async_copy(ei2_hbm.at[0, pl.ds(w * EPW, EPW)], sidx_v, sem_i)
    cp_d = pltpu.async_copy(ei2_hbm.at[1, pl.ds(w * EPW, EPW)], didx_v, sem_i)
    cp_s.wait()
    cp_d.wait()

    for b in range(NBUF):
        pltpu.async_copy(g_hbm.at[sidx_v.at[pl.ds(b * EPC, EPC)]],
                         bufs[b], sems[b])

    def it_body(it, _):
        for b in range(NBUF):
            k = it * NBUF + b
            pltpu.make_async_copy(g_hbm.at[sidx_v.at[pl.ds(k * EPC, EPC)]],
                                  bufs[b], sems[b]).wait()
            pltpu.sync_copy(bufs[b],
                            acc_sh.at[didx_v.at[pl.ds(k * EPC, EPC)]],
                            add=True)

            @pl.when(it < NCHK // NBUF - 1)
            def _():
                pltpu.async_copy(
                    g_hbm.at[sidx_v.at[pl.ds((k + NBUF) * EPC, EPC)]],
                    bufs[b], sems[b])

        return 0

    lax.fori_loop(0, NCHK // NBUF, it_body, 0)

    plsc.subcore_barrier()
    pltpu.sync_copy(acc_sh.at[pl.ds(s * RPS, RPS)],
                    out_hbm.at[c, pl.ds(s * RPS, RPS), pl.ds(0, HIDDEN)])


def _dinv(dp_ref):
    return lax.rsqrt(dp_ref[0, :N_NODES] + dp_ref[1, :N_NODES] + 1.0)


def _tc1_body(x_ref, w1_ref, dp_ref, g1_ref):
    h = jnp.dot(x_ref[...], w1_ref[...], preferred_element_type=jnp.float32)
    g1_ref[:, pl.ds(0, HIDDEN)] = h * _dinv(dp_ref)[:, None]


def _tc2_body(agg_ref, g1_ref, dp_ref, b1_ref, w2_ref, g2_ref):
    dinv = _dinv(dp_ref)
    a = (agg_ref[0, :N_NODES, :HIDDEN] + agg_ref[1, :N_NODES, :HIDDEN]
         + g1_ref[:, :HIDDEN])
    h = jnp.maximum(a * dinv[:, None] + b1_ref[...], 0.0)
    h2 = jnp.dot(h, w2_ref[...], preferred_element_type=jnp.float32)
    g2_ref[:, pl.ds(0, HIDDEN)] = h2 * dinv[:, None]


def _tc3_body(agg_ref, g2_ref, dp_ref, b2_ref, out_ref):
    dinv = _dinv(dp_ref)
    a = (agg_ref[0, :N_NODES, :HIDDEN] + agg_ref[1, :N_NODES, :HIDDEN]
         + g2_ref[:, :HIDDEN])
    o = a * dinv[:, None] + b2_ref[...]
    m = jnp.max(o, axis=1, keepdims=True)
    e = jnp.exp(o - m)
    out_ref[...] = (o - m) - jnp.log(jnp.sum(e, axis=1, keepdims=True))


_WIDE_F32 = jax.ShapeDtypeStruct((N_NODES, D_FEAT), jnp.float32)


def kernel(x, edge_index, W1, b1, W2, b2):
    ei = edge_index.astype(jnp.int32)
    npad = NEP - N_EDGES
    pad = jnp.arange(npad, dtype=jnp.int32)
    src8 = jnp.concatenate([ei[0] * 8, pad * 8])
    dstp = jnp.concatenate([ei[1], N_NODES + pad % (NP - N_NODES)])
    ei2 = jnp.stack([src8, dstp])
    zd = jnp.zeros((NP,), jnp.float32)
    z = jnp.zeros((NP, HIDDEN), jnp.float32)
    b1r = b1.reshape(1, HIDDEN)
    b2r = b2.reshape(1, HIDDEN)

    dp = _deg_kernel(ei2, zd)
    g1 = pl.pallas_call(_tc1_body, out_shape=_WIDE_F32)(x, W1, dp)
    agg1 = _agg_kernel(g1.reshape(N_NODES * 8, HIDDEN), ei2, z)
    g2 = pl.pallas_call(_tc2_body, out_shape=_WIDE_F32)(agg1, g1, dp, b1r, W2)
    agg2 = _agg_kernel(g2.reshape(N_NODES * 8, HIDDEN), ei2, z)
    out = pl.pallas_call(
        _tc3_body,
        out_shape=jax.ShapeDtypeStruct((N_NODES, HIDDEN), jnp.float32),
    )(agg2, g2, dp, b2r)
    return out
